# baseline (speedup 1.0000x reference)
import functools

import jax
import jax.numpy as jnp
from jax import lax
from jax.experimental import pallas as pl
from jax.experimental.pallas import tpu as pltpu
from jax.experimental.pallas import tpu_sc as plsc

_IMAGE = 64
_PSIZE = 8
_NP = 15
_BATCH = 4
_CHANNELS = 64
_BC = _BATCH * _CHANNELS
_NPATCH = _NP * _NP
_NPIX = _IMAGE * _IMAGE
_QUART = _NPIX // 4
_LANES = 16
_TSTRIDE = 1029


def _sc_core_counts():
    try:
        info = plsc.get_sparse_core_info()
        return info.num_cores, info.num_subcores
    except Exception:
        return 2, 16


@functools.cache
def _make_sc_kernel():
    nc, ns = _sc_core_counts()
    mesh = plsc.VectorSubcoreMesh(core_axis_name="c", subcore_axis_name="s")

    @functools.partial(
        pl.kernel,
        mesh=mesh,
        out_type=jax.ShapeDtypeStruct((_BC, _NPIX), jnp.float32),
        compiler_params=pltpu.CompilerParams(
            needs_layout_passes=False, use_tc_tiling_on_sc=False
        ),
        scratch_types=[
            pltpu.VMEM((2, _NP, 8, 8, 2 * _LANES), jnp.float32),
            pltpu.VMEM((2, _QUART, _LANES), jnp.float32),
            pltpu.VMEM((2 * _LANES, _TSTRIDE), jnp.float32),
            pltpu.SemaphoreType.DMA((2,)),
        ],
    )
    def k(x_hbm, out_hbm, slab2, obuf, tbuf, sem):
        wid = lax.axis_index("s") * nc + lax.axis_index("c")
        gg = wid // 4
        q = wid % 4
        ghi = gg // 4
        glo = gg % 4

        zeros = jnp.zeros((_LANES,), jnp.float32)

        def slab_copy(a, buf):
            return pltpu.make_async_copy(
                x_hbm.at[
                    pl.ds(a * _NP, _NP),
                    :,
                    ghi,
                    :,
                    pl.ds(glo * 2 * _LANES, 2 * _LANES),
                ],
                slab2.at[buf],
                sem.at[buf],
            )

        a_lead = jnp.maximum(4 * q - 1, 0)
        a_trail = jnp.minimum(4 * q + 3, 14)

        slab_copy(a_lead, 0).start()
        slab_copy(4 * q, 1).start()

        for half in range(2):

            def zbody(p_, _, half=half):
                obuf[half, p_, :] = zeros
                return 0

            lax.fori_loop(0, _QUART, zbody, 0, unroll=8)

        def accum(slab, b, base, i_list, ry_vecs):
            for half in range(2):
                for blk in range(0, len(i_list), 2):
                    pair = i_list[blk:blk + 2]
                    vals = [
                        slab[b, i, j, pl.ds(half * _LANES, _LANES)]
                        * ry_vecs[i]
                        for i in pair
                        for j in range(_PSIZE)
                    ]
                    n = 0
                    for i in pair:
                        for j in range(_PSIZE):
                            v = vals[n]
                            n += 1
                            if isinstance(b, int) and (
                                (b == 0 and j < 4)
                                or (b == _NP - 1 and j >= 4)
                            ):
                                v = v + v
                            plsc.addupdate(
                                obuf.at[half, base + i * _IMAGE + 4 * b + j],
                                v,
                            )

        def emit_block(slab, base, i_list, ry_vecs):
            accum(slab, 0, base, i_list, ry_vecs)

            def bbody(b, _):
                accum(slab, b, base, i_list, ry_vecs)
                return 0

            lax.fori_loop(1, _NP - 1, bbody, 0)
            accum(slab, _NP - 1, base, i_list, ry_vecs)

        quarter = jnp.float32(0.25)
        ry_const = {i: jnp.broadcast_to(quarter, (_LANES,)) for i in range(8)}

        slab_copy(a_lead, 0).wait()

        @pl.when(q > 0)
        def _():
            emit_block(slab2.at[0], -256, [4, 5, 6, 7], ry_const)

        slab_copy(4 * q + 1, 0).start()

        def main_body(s, _):
            a = 4 * q + s - 1
            buf = s % 2
            slab_copy(a, buf).wait()
            base = (4 * a) * _IMAGE - q * _QUART
            ry_vecs = {}
            for i in range(8):
                y = 4 * a + i
                ry = jnp.where(
                    (y < 4) | (y >= 60), jnp.float32(0.5), jnp.float32(0.25)
                )
                ry_vecs[i] = jnp.broadcast_to(ry, (_LANES,))
            emit_block(slab2.at[buf], base, list(range(8)), ry_vecs)

            @pl.when(s <= 2)
            def _():
                slab_copy(jnp.where(s == 2, a_trail, a + 2), buf).start()

            return 0

        lax.fori_loop(1, 4, main_body, 0)

        slab_copy(a_trail, 0).wait()

        @pl.when(q < 3)
        def _():
            emit_block(slab2.at[0], 768, [0, 1, 2, 3], ry_const)

        for half in range(2):
            rows = lax.iota(jnp.int32, _LANES) + half * _LANES

            def tbody(pix, _, rows=rows, half=half):
                v = obuf[half, pix, :]
                plsc.store_scatter(
                    tbuf, [rows, jnp.broadcast_to(pix, (_LANES,))], v
                )
                return 0

            lax.fori_loop(0, _QUART, tbody, 0, unroll=8)

        pltpu.sync_copy(
            tbuf.at[:, pl.ds(0, _QUART)],
            out_hbm.at[
                pl.ds(gg * 2 * _LANES, 2 * _LANES), pl.ds(q * _QUART, _QUART)
            ],
        )

    return k


def kernel(input_data):
    x5 = (
        input_data.transpose(1, 2, 0)
        .reshape(_NPATCH, 8, 8, 2, 128)
        .transpose(0, 1, 3, 2, 4)
    )
    out = _make_sc_kernel()(x5)
    return out.reshape(_BATCH, _CHANNELS, _IMAGE, _IMAGE)
